# Initial kernel scaffold; baseline (speedup 1.0000x reference)
import jax
import jax.numpy as jnp
from jax import lax
from jax.experimental import pallas as pl
from jax.experimental.pallas import tpu as pltpu
from jax.experimental.pallas import tpu_sc as plsc

N_NODES = 66125
N_PAD = 66560
H = 2
C = 64

NC, NS = 2, 16
BLK = 5888
NBATCH = BLK // 128
NBLK = 12
EB_SCAN = NBLK * BLK
EPAD = NS * EB_SCAN
CH = 12288
NCH_SC = 3
NACC = CH * NCH_SC * NC
ROWS_T = CH // NS
PAD_DST = 1 << 30


def _prep_body(x_ref, w_ref, asrc_ref, adst_ref, xa_ref, at_ref):
    xb = x_ref[...]
    w = w_ref[...]
    xw = lax.dot_general(xb, w, (((1,), (1,)), ((), ())),
                         preferred_element_type=jnp.float32)
    a_s = asrc_ref[...].reshape(H, C)
    a_d = adst_ref[...].reshape(H, C)
    cols = []
    for tab in (a_s, a_d):
        for h in range(H):
            v = jnp.sum(xw[:, h * C:(h + 1) * C] * tab[h][None, :], axis=1)
            cols.append(v.reshape(-1, 1))
    z12 = jnp.zeros((xb.shape[0], 12), jnp.float32)
    z14 = jnp.zeros((xb.shape[0], 14), jnp.float32)
    xa_ref[...] = jnp.concatenate([xb, cols[0], cols[1], z14], axis=1)
    at_ref[...] = jnp.concatenate([cols[2], cols[3], z14], axis=1)


def _prep(x, W, att_src, att_dst):
    grid = (N_PAD // 512,)
    return pl.pallas_call(
        _prep_body,
        grid=grid,
        in_specs=[
            pl.BlockSpec((512, 64), lambda i: (i, 0)),
            pl.BlockSpec((128, 64), lambda i: (0, 0)),
            pl.BlockSpec((1, H, C), lambda i: (0, 0, 0)),
            pl.BlockSpec((1, H, C), lambda i: (0, 0, 0)),
        ],
        out_specs=[
            pl.BlockSpec((512, 80), lambda i: (i, 0)),
            pl.BlockSpec((512, 16), lambda i: (i, 0)),
        ],
        out_shape=[
            jax.ShapeDtypeStruct((N_PAD, 80), jnp.float32),
            jax.ShapeDtypeStruct((N_PAD, 16), jnp.float32),
        ],
    )(x, W, att_src, att_dst)


def _edge_body(xa_hbm, at_hbm, src_hbm, dst_hbm, acc_hbm,
               src_v, dst_v, sidx, gdidx, sdidx,
               xabuf, atbuf, w0buf, w1buf, srows, acc_sp, gsem):
    cid = lax.axis_index("c")
    tid = lax.axis_index("s")
    iota16 = lax.iota(jnp.int32, (16,))
    zero16 = jnp.zeros((16,), jnp.float32)

    def chunk_body(j, _):
        lo = (cid * NCH_SC + j) * CH

        def zrow(r, _):
            for c in range(9):
                srows[r, pl.ds(c * 16, 16)] = zero16
            return 0
        lax.fori_loop(0, 128, zrow, 0)

        def zacc(t, _):
            pltpu.sync_copy(srows,
                            acc_sp.at[pl.ds(tid * ROWS_T + t * 128, 128), :])
            return 0
        lax.fori_loop(0, ROWS_T // 128, zacc, 0)
        plsc.subcore_barrier()

        def block_body(blk, _):
            base = tid * EB_SCAN + blk * BLK
            pltpu.sync_copy(src_hbm.at[pl.ds(base, BLK)], src_v)
            pltpu.sync_copy(dst_hbm.at[pl.ds(base, BLK)], dst_v)

            def batch_body(b, _):
                def grp(q, _):
                    off = b * 128 + q * 16
                    s = src_v[pl.ds(off, 16)]
                    d = dst_v[pl.ds(off, 16)]
                    valid = (d >= lo) & (d < lo + CH)
                    sidx[0, pl.ds(q * 16, 16)] = s
                    gdidx[0, pl.ds(q * 16, 16)] = jnp.where(d < N_NODES, d, 0)
                    sdidx[0, pl.ds(q * 16, 16)] = jnp.where(valid, d - lo, -1)
                    return 0
                lax.fori_loop(0, 8, grp, 0)

                cg1 = pltpu.async_copy(
                    xa_hbm.at[plsc.Indices(sidx.at[0])], xabuf, gsem)
                cg2 = pltpu.async_copy(
                    at_hbm.at[plsc.Indices(gdidx.at[0])], atbuf, gsem)
                cg1.wait()
                cg2.wait()

                def wgrp(q, _):
                    rows = q * 16 + iota16
                    c64 = jnp.full((16,), 64, jnp.int32)
                    c65 = jnp.full((16,), 65, jnp.int32)
                    c0 = jnp.full((16,), 0, jnp.int32)
                    c1 = jnp.full((16,), 1, jnp.int32)
                    as0 = plsc.load_gather(xabuf, [rows, c64])
                    as1 = plsc.load_gather(xabuf, [rows, c65])
                    ad0 = plsc.load_gather(atbuf, [rows, c0])
                    ad1 = plsc.load_gather(atbuf, [rows, c1])
                    sv = sdidx[0, pl.ds(q * 16, 16)]
                    ok = sv >= 0
                    a0 = as0 + ad0
                    a0 = jnp.where(a0 >= 0.0, a0, 0.2 * a0)
                    w0 = jnp.where(ok, jnp.exp(a0), 0.0)
                    a1 = as1 + ad1
                    a1 = jnp.where(a1 >= 0.0, a1, 0.2 * a1)
                    w1 = jnp.where(ok, jnp.exp(a1), 0.0)
                    w0buf[0, pl.ds(q * 16, 16)] = w0
                    w1buf[0, pl.ds(q * 16, 16)] = w1
                    return 0
                lax.fori_loop(0, 8, wgrp, 0)

                def sedge(e, _):
                    w0 = w0buf[0, e]
                    w1 = w1buf[0, e]
                    for r in range(4):
                        xv = xabuf[e, pl.ds(r * 16, 16)]
                        srows[e, pl.ds(r * 16, 16)] = xv * w0
                        srows[e, pl.ds(64 + r * 16, 16)] = xv * w1
                    tail = jnp.where(iota16 == 0, w0,
                                     jnp.where(iota16 == 1, w1, 0.0))
                    srows[e, pl.ds(128, 16)] = tail
                    return 0
                lax.fori_loop(0, 128, sedge, 0)

                pltpu.sync_copy(
                    srows,
                    acc_sp.at[plsc.Indices(sdidx.at[0], ignored_value=-1)],
                    add=True)
                return 0
            lax.fori_loop(0, NBATCH, batch_body, 0)
            return 0
        lax.fori_loop(0, NBLK, block_body, 0)
        plsc.subcore_barrier()

        def dump(t, _):
            r = tid * ROWS_T + t * 128
            pltpu.sync_copy(acc_sp.at[pl.ds(r, 128), :],
                            acc_hbm.at[pl.ds(lo + r, 128), :])
            return 0
        lax.fori_loop(0, ROWS_T // 128, dump, 0)
        return 0
    lax.fori_loop(0, NCH_SC, chunk_body, 0)


def _edge_phase(xa, at, src_pad, dst_pad):
    mesh = plsc.VectorSubcoreMesh(core_axis_name="c", subcore_axis_name="s",
                                  num_cores=NC, num_subcores=NS)
    fn = pl.kernel(
        _edge_body,
        out_type=jax.ShapeDtypeStruct((NACC, 144), jnp.float32),
        mesh=mesh,
        scratch_types=[
            pltpu.VMEM((BLK,), jnp.int32),
            pltpu.VMEM((BLK,), jnp.int32),
            pltpu.VMEM((1, 128), jnp.int32),
            pltpu.VMEM((1, 128), jnp.int32),
            pltpu.VMEM((1, 128), jnp.int32),
            pltpu.VMEM((128, 80), jnp.float32),
            pltpu.VMEM((128, 16), jnp.float32),
            pltpu.VMEM((1, 128), jnp.float32),
            pltpu.VMEM((1, 128), jnp.float32),
            pltpu.VMEM((128, 144), jnp.float32),
            pltpu.VMEM_SHARED((CH, 144), jnp.float32),
            pltpu.SemaphoreType.DMA,
        ],
    )
    return fn(xa, at, src_pad, dst_pad)


def _final_body(acc_ref, w_ref, b_ref, out_ref):
    a = acc_ref[...]
    w = w_ref[...]
    outs = []
    for h in range(H):
        d = a[:, 128 + h:129 + h] + 1e-16
        m = a[:, h * C:(h + 1) * C] / d
        y = lax.dot_general(m, w[h * C:(h + 1) * C, :],
                            (((1,), (1,)), ((), ())),
                            preferred_element_type=jnp.float32)
        outs.append(y)
    out_ref[...] = jnp.concatenate(outs, axis=1) + b_ref[...][None, :]


def _final(acc, W, bias):
    grid = (N_PAD // 512,)
    return pl.pallas_call(
        _final_body,
        grid=grid,
        in_specs=[
            pl.BlockSpec((512, 144), lambda i: (i, 0)),
            pl.BlockSpec((128, 64), lambda i: (0, 0)),
            pl.BlockSpec((128,), lambda i: (0,)),
        ],
        out_specs=pl.BlockSpec((512, 128), lambda i: (i, 0)),
        out_shape=jax.ShapeDtypeStruct((N_NODES, H * C), jnp.float32),
    )(acc, W, bias)


def kernel(x, W, att_src, att_dst, bias, edge_index):
    n = x.shape[0]
    loop = jnp.arange(n, dtype=edge_index.dtype)
    src = jnp.concatenate([edge_index[0], loop])
    dst = jnp.concatenate([edge_index[1], loop])
    npad = EPAD - src.shape[0]
    src_pad = jnp.concatenate([src, jnp.zeros((npad,), jnp.int32)])
    dst_pad = jnp.concatenate([dst, jnp.full((npad,), PAD_DST, jnp.int32)])
    xa, at = _prep(x, W, att_src, att_dst)
    acc = _edge_phase(xa, at, src_pad, dst_pad)
    return _final(acc, W, bias)


# speedup vs baseline: 10.7753x; 10.7753x over previous
import jax
import jax.numpy as jnp
from jax import lax
from jax.experimental import pallas as pl
from jax.experimental.pallas import tpu as pltpu
from jax.experimental.pallas import tpu_sc as plsc

N_NODES = 66125
N_PAD = 66560
H = 2
C = 64

NC, NS = 2, 16
BLK = 5888
NBATCH = BLK // 128
NBLK = 12
EB_SCAN = NBLK * BLK
EPAD = NS * EB_SCAN
CH = 9216
NCH_SC = 4
NACC = CH * NCH_SC * NC
ROWS_T = CH // NS
PAD_DST = 1 << 30


def _prep_body(x_ref, w_ref, asrc_ref, adst_ref, as_ref, ad_ref):
    xb = x_ref[...]
    w = w_ref[...]
    xw = lax.dot_general(xb, w, (((1,), (1,)), ((), ())),
                         preferred_element_type=jnp.float32)
    a_s = asrc_ref[...].reshape(H, C)
    a_d = adst_ref[...].reshape(H, C)
    cols = []
    for tab in (a_s, a_d):
        for h in range(H):
            v = jnp.sum(xw[:, h * C:(h + 1) * C] * tab[h][None, :], axis=1)
            cols.append(v.reshape(-1, 1))
    as_ref[...] = jnp.concatenate([cols[0], cols[1]] * 8, axis=1)
    ad_ref[...] = jnp.concatenate([cols[2], cols[3]] * 8, axis=1)


def _prep(x, W, att_src, att_dst):
    grid = (N_PAD // 512,)
    return pl.pallas_call(
        _prep_body,
        grid=grid,
        in_specs=[
            pl.BlockSpec((512, 64), lambda i: (i, 0)),
            pl.BlockSpec((128, 64), lambda i: (0, 0)),
            pl.BlockSpec((1, H, C), lambda i: (0, 0, 0)),
            pl.BlockSpec((1, H, C), lambda i: (0, 0, 0)),
        ],
        out_specs=[
            pl.BlockSpec((512, 16), lambda i: (i, 0)),
            pl.BlockSpec((512, 16), lambda i: (i, 0)),
        ],
        out_shape=[
            jax.ShapeDtypeStruct((N_PAD, 16), jnp.float32),
            jax.ShapeDtypeStruct((N_PAD, 16), jnp.float32),
        ],
    )(x, W, att_src, att_dst)


def _edge_body(x_hbm, as_hbm, ad_hbm, src_hbm, dst_hbm, acc_hbm,
               src_v, dst_v, sidx, gdidx, sdidx,
               xbuf, asbuf, adbuf, srows, acc_sp, gsem):
    cid = lax.axis_index("c")
    tid = lax.axis_index("s")
    iota16 = lax.iota(jnp.int32, 16)
    zero16 = jnp.zeros((16,), jnp.float32)

    def chunk_body(j, _):
        lo = (cid * NCH_SC + j) * CH

        def zrow(r, _):
            for c in range(9):
                srows[r, pl.ds(c * 16, 16)] = zero16
            return 0
        lax.fori_loop(0, 128, zrow, 0)

        def zacc(t, _):
            pltpu.sync_copy(srows.at[pl.ds(0, 64), :],
                            acc_sp.at[pl.ds(tid * ROWS_T + t * 64, 64), :])
            return 0
        lax.fori_loop(0, ROWS_T // 64, zacc, 0)
        plsc.subcore_barrier()

        def block_body(blk, _):
            base = tid * EB_SCAN + blk * BLK
            pltpu.sync_copy(src_hbm.at[pl.ds(base, BLK)], src_v)
            pltpu.sync_copy(dst_hbm.at[pl.ds(base, BLK)], dst_v)

            def batch_body(b, _):
                def grp(q, _):
                    off = b * 128 + q * 16
                    s = src_v[pl.ds(off, 16)]
                    d = dst_v[pl.ds(off, 16)]
                    valid = (d >= lo) & (d < lo + CH)
                    sidx[0, pl.ds(q * 16, 16)] = s
                    gdidx[0, pl.ds(q * 16, 16)] = jnp.where(d < N_NODES, d, 0)
                    sdidx[0, pl.ds(q * 16, 16)] = jnp.where(valid, d - lo, -1)
                    return 0
                lax.fori_loop(0, 8, grp, 0)

                cg1 = pltpu.async_copy(
                    x_hbm.at[plsc.Indices(sidx.at[0])], xbuf, gsem)
                cg2 = pltpu.async_copy(
                    as_hbm.at[plsc.Indices(sidx.at[0])], asbuf, gsem)
                cg3 = pltpu.async_copy(
                    ad_hbm.at[plsc.Indices(gdidx.at[0])], adbuf, gsem)
                cg1.wait()
                cg2.wait()
                cg3.wait()

                def sgrp(q, _):
                    for l in range(16):
                        e = q * 16 + l
                        av = asbuf[e, :] + adbuf[e, :]
                        av = jnp.where(av >= 0.0, av, 0.2 * av)
                        wv = jnp.exp(av)
                        w0 = wv[0]
                        w1 = wv[1]
                        for r in range(4):
                            xv = xbuf[e, pl.ds(r * 16, 16)]
                            srows[e, pl.ds(r * 16, 16)] = xv * w0
                            srows[e, pl.ds(64 + r * 16, 16)] = xv * w1
                        srows[e, pl.ds(128, 16)] = jnp.where(iota16 < 2, wv, 0.0)
                    return 0
                lax.fori_loop(0, 8, sgrp, 0)

                pltpu.sync_copy(
                    srows,
                    acc_sp.at[plsc.Indices(sdidx.at[0], ignored_value=-1)],
                    add=True)
                return 0
            lax.fori_loop(0, NBATCH, batch_body, 0)
            return 0
        lax.fori_loop(0, NBLK, block_body, 0)
        plsc.subcore_barrier()

        def dump(t, _):
            r = tid * ROWS_T + t * 64
            pltpu.sync_copy(acc_sp.at[pl.ds(r, 64), :],
                            acc_hbm.at[pl.ds(lo + r, 64), :])
            return 0
        lax.fori_loop(0, ROWS_T // 64, dump, 0)
        return 0
    lax.fori_loop(0, NCH_SC, chunk_body, 0)


def _edge_phase(x, as16, ad16, src_pad, dst_pad):
    mesh = plsc.VectorSubcoreMesh(core_axis_name="c", subcore_axis_name="s",
                                  num_cores=NC, num_subcores=NS)
    fn = pl.kernel(
        _edge_body,
        out_type=jax.ShapeDtypeStruct((NACC, 144), jnp.float32),
        mesh=mesh,
        compiler_params=pltpu.CompilerParams(use_tc_tiling_on_sc=False),
        scratch_types=[
            pltpu.VMEM((BLK,), jnp.int32),
            pltpu.VMEM((BLK,), jnp.int32),
            pltpu.VMEM((1, 128), jnp.int32),
            pltpu.VMEM((1, 128), jnp.int32),
            pltpu.VMEM((1, 128), jnp.int32),
            pltpu.VMEM((128, 64), jnp.float32),
            pltpu.VMEM((128, 16), jnp.float32),
            pltpu.VMEM((128, 16), jnp.float32),
            pltpu.VMEM((128, 144), jnp.float32),
            pltpu.VMEM_SHARED((CH, 144), jnp.float32),
            pltpu.SemaphoreType.DMA,
        ],
    )
    return fn(x, as16, ad16, src_pad, dst_pad)


def _final_body(acc_ref, w_ref, b_ref, out_ref):
    a = acc_ref[...]
    w = w_ref[...]
    outs = []
    for h in range(H):
        d = a[:, 128 + h:129 + h] + 1e-16
        m = a[:, h * C:(h + 1) * C] / d
        y = lax.dot_general(m, w[h * C:(h + 1) * C, :],
                            (((1,), (1,)), ((), ())),
                            preferred_element_type=jnp.float32)
        outs.append(y)
    out_ref[...] = jnp.concatenate(outs, axis=1) + b_ref[...][None, :]


def _final(acc, W, bias):
    grid = (N_PAD // 512,)
    return pl.pallas_call(
        _final_body,
        grid=grid,
        in_specs=[
            pl.BlockSpec((512, 144), lambda i: (i, 0)),
            pl.BlockSpec((128, 64), lambda i: (0, 0)),
            pl.BlockSpec((128,), lambda i: (0,)),
        ],
        out_specs=pl.BlockSpec((512, 128), lambda i: (i, 0)),
        out_shape=jax.ShapeDtypeStruct((N_NODES, H * C), jnp.float32),
    )(acc, W, bias)


def kernel(x, W, att_src, att_dst, bias, edge_index):
    n = x.shape[0]
    loop = jnp.arange(n, dtype=edge_index.dtype)
    src = jnp.concatenate([edge_index[0], loop])
    dst = jnp.concatenate([edge_index[1], loop])
    npad = EPAD - src.shape[0]
    src_pad = jnp.concatenate([src, jnp.zeros((npad,), jnp.int32)])
    dst_pad = jnp.concatenate([dst, jnp.full((npad,), PAD_DST, jnp.int32)])
    as16, ad16 = _prep(x, W, att_src, att_dst)
    acc = _edge_phase(x, as16, ad16, src_pad, dst_pad)
    return _final(acc, W, bias)


# speedup vs baseline: 41.8700x; 3.8857x over previous
import jax
import jax.numpy as jnp
from jax import lax
from jax.experimental import pallas as pl
from jax.experimental.pallas import tpu as pltpu
from jax.experimental.pallas import tpu_sc as plsc

N_NODES = 66125
N_PAD = 66560
H = 2
C = 64

NC, NS = 2, 16
BLK = 5888
NBATCH = BLK // 128
NBLK = 12
EB_SCAN = NBLK * BLK
EPAD = NS * EB_SCAN
CH = 8192
NCHUNK = 9
NACC = CH * NCHUNK
ROWS_T = CH // NS
PAD_DST = 1 << 30


def _prep_body(x_ref, w_ref, asrc_ref, adst_ref, as_ref, ad_ref):
    xb = x_ref[...]
    w = w_ref[...]
    xw = lax.dot_general(xb, w, (((1,), (1,)), ((), ())),
                         preferred_element_type=jnp.float32)
    a_s = asrc_ref[...].reshape(H, C)
    a_d = adst_ref[...].reshape(H, C)
    cols = []
    for tab in (a_s, a_d):
        for h in range(H):
            v = jnp.sum(xw[:, h * C:(h + 1) * C] * tab[h][None, :], axis=1)
            cols.append(v.reshape(-1, 1))
    as_ref[...] = jnp.concatenate([cols[0], cols[1]] * 8, axis=1)
    ad_ref[...] = jnp.concatenate([cols[2], cols[3]] * 8, axis=1)


def _prep(x, W, att_src, att_dst):
    grid = (N_PAD // 512,)
    return pl.pallas_call(
        _prep_body,
        grid=grid,
        in_specs=[
            pl.BlockSpec((512, 64), lambda i: (i, 0)),
            pl.BlockSpec((128, 64), lambda i: (0, 0)),
            pl.BlockSpec((1, H, C), lambda i: (0, 0, 0)),
            pl.BlockSpec((1, H, C), lambda i: (0, 0, 0)),
        ],
        out_specs=[
            pl.BlockSpec((512, 16), lambda i: (i, 0)),
            pl.BlockSpec((512, 16), lambda i: (i, 0)),
        ],
        out_shape=[
            jax.ShapeDtypeStruct((N_PAD, 16), jnp.float32),
            jax.ShapeDtypeStruct((N_PAD, 16), jnp.float32),
        ],
    )(x, W, att_src, att_dst)


def _edge_body(x_hbm, as_hbm, ad_hbm, src_hbm, dst_hbm, acc_hbm,
               src_v, dst_v, csrc, cdst, sdidx,
               xbuf, asbuf, adbuf, srows, acc_sp, gsem):
    cid = lax.axis_index("c")
    tid = lax.axis_index("s")
    iota16 = lax.iota(jnp.int32, 16)
    zero16 = jnp.zeros((16,), jnp.float32)

    def chunk_body(j, _):
        lo = (cid * 5 + j) * CH

        def zrow(r, _):
            for c in range(9):
                srows[r, pl.ds(c * 16, 16)] = zero16
            return 0
        lax.fori_loop(0, 128, zrow, 0)

        def zacc(t, _):
            pltpu.sync_copy(srows.at[pl.ds(0, 64), :],
                            acc_sp.at[pl.ds(tid * ROWS_T + t * 64, 64), :])
            return 0
        lax.fori_loop(0, ROWS_T // 64, zacc, 0)
        plsc.subcore_barrier()

        def block_body(blk, _):
            base = tid * EB_SCAN + blk * BLK
            pltpu.sync_copy(src_hbm.at[pl.ds(base, BLK)], src_v)
            pltpu.sync_copy(dst_hbm.at[pl.ds(base, BLK)], dst_v)

            def filt(g, k):
                s = src_v[pl.ds(g * 16, 16)]
                d = dst_v[pl.ds(g * 16, 16)]
                valid = (d >= lo) & (d < lo + CH)
                keys = jnp.where(valid, iota16, 16)
                _, ssort = plsc.sort_key_val(keys, s)
                _, dsort = plsc.sort_key_val(keys, d)
                csrc[pl.ds(k, 16)] = ssort
                cdst[pl.ds(k, 16)] = dsort
                return k + plsc.all_reduce_population_count(valid)[0]
            k = lax.fori_loop(0, BLK // 16, filt, jnp.int32(0))

            kp = ((k + 127) // 128) * 128
            zero16i = jnp.zeros((16,), jnp.int32)

            def padv(v, _):
                lane = v * 16 + iota16
                keep = lane < k
                sv = csrc[pl.ds(v * 16, 16)]
                dv = cdst[pl.ds(v * 16, 16)]
                csrc[pl.ds(v * 16, 16)] = jnp.where(keep, sv, zero16i)
                cdst[pl.ds(v * 16, 16)] = jnp.where(keep, dv, zero16i)
                return 0
            lax.fori_loop(k // 16, kp // 16, padv, 0)

            def batch_body(b, _):
                def cpy(q, _):
                    lane = b * 128 + q * 16
                    dv = cdst[pl.ds(lane, 16)]
                    sdidx[0, pl.ds(q * 16, 16)] = jnp.where(
                        lane + iota16 < k, dv - lo, -1)
                    return 0
                lax.fori_loop(0, 8, cpy, 0)

                cg1 = pltpu.async_copy(
                    x_hbm.at[plsc.Indices(csrc.at[pl.ds(b * 128, 128)])],
                    xbuf, gsem)
                cg2 = pltpu.async_copy(
                    as_hbm.at[plsc.Indices(csrc.at[pl.ds(b * 128, 128)])],
                    asbuf, gsem)
                cg3 = pltpu.async_copy(
                    ad_hbm.at[plsc.Indices(cdst.at[pl.ds(b * 128, 128)])],
                    adbuf, gsem)
                cg1.wait()
                cg2.wait()
                cg3.wait()

                def sgrp(q, _):
                    for l in range(16):
                        e = q * 16 + l
                        av = asbuf[e, :] + adbuf[e, :]
                        av = jnp.where(av >= 0.0, av, 0.2 * av)
                        wv = jnp.exp(av)
                        w0 = wv[0]
                        w1 = wv[1]
                        for r in range(4):
                            xv = xbuf[e, pl.ds(r * 16, 16)]
                            srows[e, pl.ds(r * 16, 16)] = xv * w0
                            srows[e, pl.ds(64 + r * 16, 16)] = xv * w1
                        srows[e, pl.ds(128, 16)] = jnp.where(iota16 < 2, wv, 0.0)
                    return 0
                lax.fori_loop(0, 8, sgrp, 0)

                pltpu.sync_copy(
                    srows,
                    acc_sp.at[plsc.Indices(sdidx.at[0], ignored_value=-1)],
                    add=True)
                return 0
            lax.fori_loop(0, kp // 128, batch_body, 0)
            return 0
        lax.fori_loop(0, NBLK, block_body, 0)
        plsc.subcore_barrier()

        def dump(t, _):
            r = tid * ROWS_T + t * 64
            pltpu.sync_copy(acc_sp.at[pl.ds(r, 64), :],
                            acc_hbm.at[pl.ds(lo + r, 64), :])
            return 0
        lax.fori_loop(0, ROWS_T // 64, dump, 0)
        return 0
    lax.fori_loop(0, jnp.where(cid == 0, 5, 4), chunk_body, 0)


def _edge_phase(x, as16, ad16, src_pad, dst_pad):
    mesh = plsc.VectorSubcoreMesh(core_axis_name="c", subcore_axis_name="s",
                                  num_cores=NC, num_subcores=NS)
    fn = pl.kernel(
        _edge_body,
        out_type=jax.ShapeDtypeStruct((NACC, 144), jnp.float32),
        mesh=mesh,
        compiler_params=pltpu.CompilerParams(use_tc_tiling_on_sc=False, needs_layout_passes=False),
        scratch_types=[
            pltpu.VMEM((BLK,), jnp.int32),
            pltpu.VMEM((BLK,), jnp.int32),
            pltpu.VMEM((BLK + 128,), jnp.int32),
            pltpu.VMEM((BLK + 128,), jnp.int32),
            pltpu.VMEM((1, 128), jnp.int32),
            pltpu.VMEM((128, 64), jnp.float32),
            pltpu.VMEM((128, 16), jnp.float32),
            pltpu.VMEM((128, 16), jnp.float32),
            pltpu.VMEM((128, 144), jnp.float32),
            pltpu.VMEM_SHARED((CH, 144), jnp.float32),
            pltpu.SemaphoreType.DMA,
        ],
    )
    return fn(x, as16, ad16, src_pad, dst_pad)


def _final_body(acc_ref, w_ref, b_ref, out_ref):
    a = acc_ref[...]
    w = w_ref[...]
    outs = []
    for h in range(H):
        d = a[:, 128 + h:129 + h] + 1e-16
        m = a[:, h * C:(h + 1) * C] / d
        y = lax.dot_general(m, w[h * C:(h + 1) * C, :],
                            (((1,), (1,)), ((), ())),
                            preferred_element_type=jnp.float32)
        outs.append(y)
    out_ref[...] = jnp.concatenate(outs, axis=1) + b_ref[...][None, :]


def _final(acc, W, bias):
    grid = (N_PAD // 512,)
    return pl.pallas_call(
        _final_body,
        grid=grid,
        in_specs=[
            pl.BlockSpec((512, 144), lambda i: (i, 0)),
            pl.BlockSpec((128, 64), lambda i: (0, 0)),
            pl.BlockSpec((128,), lambda i: (0,)),
        ],
        out_specs=pl.BlockSpec((512, 128), lambda i: (i, 0)),
        out_shape=jax.ShapeDtypeStruct((N_NODES, H * C), jnp.float32),
    )(acc, W, bias)


def kernel(x, W, att_src, att_dst, bias, edge_index):
    n = x.shape[0]
    loop = jnp.arange(n, dtype=edge_index.dtype)
    src = jnp.concatenate([edge_index[0], loop])
    dst = jnp.concatenate([edge_index[1], loop])
    npad = EPAD - src.shape[0]
    src_pad = jnp.concatenate([src, jnp.zeros((npad,), jnp.int32)])
    dst_pad = jnp.concatenate([dst, jnp.full((npad,), PAD_DST, jnp.int32)])
    as16, ad16 = _prep(x, W, att_src, att_dst)
    acc = _edge_phase(x, as16, ad16, src_pad, dst_pad)
    return _final(acc, W, bias)


# speedup vs baseline: 49.6516x; 1.1859x over previous
import jax
import jax.numpy as jnp
from jax import lax
from jax.experimental import pallas as pl
from jax.experimental.pallas import tpu as pltpu
from jax.experimental.pallas import tpu_sc as plsc

N_NODES = 66125
N_PAD = 66560
H = 2
C = 64

NC, NS = 2, 16
BLK = 5888
NBATCH = BLK // 128
NBLK = 12
EB_SCAN = NBLK * BLK
EPAD = NS * EB_SCAN
CH = 9216
NCHUNK = 8
NACC = CH * NCHUNK
BT = 64
ROWS_T = CH // NS
PAD_DST = 1 << 30


def _prep_body(x_ref, w_ref, asrc_ref, adst_ref, as_ref, ad_ref):
    xb = x_ref[...]
    w = w_ref[...]
    xw = lax.dot_general(xb, w, (((1,), (1,)), ((), ())),
                         preferred_element_type=jnp.float32)
    a_s = asrc_ref[...].reshape(H, C)
    a_d = adst_ref[...].reshape(H, C)
    cols = []
    for tab in (a_s, a_d):
        for h in range(H):
            v = jnp.sum(xw[:, h * C:(h + 1) * C] * tab[h][None, :], axis=1)
            cols.append(v.reshape(-1, 1))
    as_ref[...] = jnp.concatenate([cols[0], cols[1]] * 8, axis=1)
    ad_ref[...] = jnp.concatenate([cols[2], cols[3]] * 8, axis=1)


def _prep(x, W, att_src, att_dst):
    grid = (N_PAD // 512,)
    return pl.pallas_call(
        _prep_body,
        grid=grid,
        in_specs=[
            pl.BlockSpec((512, 64), lambda i: (i, 0)),
            pl.BlockSpec((128, 64), lambda i: (0, 0)),
            pl.BlockSpec((1, H, C), lambda i: (0, 0, 0)),
            pl.BlockSpec((1, H, C), lambda i: (0, 0, 0)),
        ],
        out_specs=[
            pl.BlockSpec((512, 16), lambda i: (i, 0)),
            pl.BlockSpec((512, 16), lambda i: (i, 0)),
        ],
        out_shape=[
            jax.ShapeDtypeStruct((N_PAD, 16), jnp.float32),
            jax.ShapeDtypeStruct((N_PAD, 16), jnp.float32),
        ],
    )(x, W, att_src, att_dst)


def _edge_body(x_hbm, as_hbm, ad_hbm, src_hbm, dst_hbm, acc_hbm,
               src_v, dst_v, cpack, sidx, adidx, sdidx,
               xbuf, asbuf, adbuf, srows, acc_sp, gsem, ssem):
    cid = lax.axis_index("c")
    tid = lax.axis_index("s")
    iota16 = lax.iota(jnp.int32, 16)
    zero16 = jnp.zeros((16,), jnp.float32)
    zero16i = jnp.zeros((16,), jnp.int32)
    MAXI = jnp.int32(0x7FFFFFFF)

    def chunk_body(j, _):
        lo = (cid * 4 + j) * CH

        def zrow(r, _):
            for c in range(9):
                srows[0, r, pl.ds(c * 16, 16)] = zero16
            return 0
        lax.fori_loop(0, 64, zrow, 0)

        def zacc(t, _):
            pltpu.sync_copy(srows.at[0],
                            acc_sp.at[pl.ds(tid * ROWS_T + t * 64, 64), :])
            return 0
        lax.fori_loop(0, ROWS_T // 64, zacc, 0)
        plsc.subcore_barrier()

        def block_body(blk, _):
            base = tid * EB_SCAN + blk * BLK
            pltpu.sync_copy(src_hbm.at[pl.ds(base, BLK)], src_v)
            pltpu.sync_copy(dst_hbm.at[pl.ds(base, BLK)], dst_v)

            def filt(g, k):
                s = src_v[pl.ds(g * 16, 16)]
                d = dst_v[pl.ds(g * 16, 16)]
                valid = (d >= lo) & (d < lo + CH)
                packed = jnp.where(
                    valid, lax.shift_left(s, 13) | (d - lo), MAXI)
                ks, _unused = plsc.sort_key_val(packed, packed)
                cpack[pl.ds(k, 16)] = ks
                return k + plsc.all_reduce_population_count(valid)[0]
            k = lax.fori_loop(0, BLK // 16, filt, jnp.int32(0))
            kp = ((k + 63) // 64) * 64
            nb = kp // 64

            def padv(v, _):
                lane = v * 16 + iota16
                pv = cpack[pl.ds(v * 16, 16)]
                cpack[pl.ds(v * 16, 16)] = jnp.where(lane < k, pv, zero16i)
                return 0
            lax.fori_loop(k // 16, kp // 16, padv, 0)

            def batch_body(b, _):
                par = b % 2

                def cpy(q, _):
                    v = cpack[pl.ds(b * 64 + q * 16, 16)]
                    s = lax.shift_right_logical(v, 13)
                    drel = v & 8191
                    lane = b * 64 + q * 16 + iota16
                    sidx[0, pl.ds(q * 16, 16)] = s
                    adidx[0, pl.ds(q * 16, 16)] = drel + lo
                    sdidx[par, pl.ds(q * 16, 16)] = jnp.where(
                        lane < k, drel, CH)
                    return 0
                lax.fori_loop(0, 4, cpy, 0)

                cg1 = pltpu.async_copy(
                    x_hbm.at[plsc.Indices(sidx.at[0])], xbuf, gsem)
                cg2 = pltpu.async_copy(
                    as_hbm.at[plsc.Indices(sidx.at[0])], asbuf, gsem)
                cg3 = pltpu.async_copy(
                    ad_hbm.at[plsc.Indices(adidx.at[0])], adbuf, gsem)
                cg1.wait()
                cg2.wait()
                cg3.wait()

                @pl.when(b >= 2)
                def _():
                    pltpu.make_async_copy(
                        srows.at[0],
                        acc_sp.at[pl.ds(0, 64), :], ssem).wait()

                def sgrp(q, _):
                    for l in range(16):
                        e = q * 16 + l
                        av = asbuf[e, :] + adbuf[e, :]
                        av = jnp.where(av >= 0.0, av, 0.2 * av)
                        wv = jnp.exp(av)
                        w0 = wv[0]
                        w1 = wv[1]
                        for r in range(4):
                            xv = xbuf[e, pl.ds(r * 16, 16)]
                            srows[par, e, pl.ds(r * 16, 16)] = xv * w0
                            srows[par, e, pl.ds(64 + r * 16, 16)] = xv * w1
                        srows[par, e, pl.ds(128, 16)] = jnp.where(
                            iota16 < 2, wv, 0.0)
                    return 0
                lax.fori_loop(0, 4, sgrp, 0)

                pltpu.async_copy(
                    srows.at[par],
                    acc_sp.at[plsc.Indices(sdidx.at[par])], ssem,
                    add=True)
                return 0
            lax.fori_loop(0, nb, batch_body, 0)

            def drain(i, _):
                pltpu.make_async_copy(
                    srows.at[0], acc_sp.at[pl.ds(0, 64), :], ssem).wait()
                return 0
            lax.fori_loop(0, jnp.minimum(nb, 2), drain, 0)
            return 0
        lax.fori_loop(0, NBLK, block_body, 0)
        plsc.subcore_barrier()

        def dump(t, _):
            r = tid * ROWS_T + t * 64
            pltpu.sync_copy(acc_sp.at[pl.ds(r, 64), :],
                            acc_hbm.at[pl.ds(lo + r, 64), :])
            return 0
        lax.fori_loop(0, ROWS_T // 64, dump, 0)
        return 0
    lax.fori_loop(0, 4, chunk_body, 0)


def _edge_phase(x, as16, ad16, src_pad, dst_pad):
    mesh = plsc.VectorSubcoreMesh(core_axis_name="c", subcore_axis_name="s",
                                  num_cores=NC, num_subcores=NS)
    fn = pl.kernel(
        _edge_body,
        out_type=jax.ShapeDtypeStruct((NACC, 144), jnp.float32),
        mesh=mesh,
        compiler_params=pltpu.CompilerParams(use_tc_tiling_on_sc=False, needs_layout_passes=False),
        scratch_types=[
            pltpu.VMEM((BLK,), jnp.int32),
            pltpu.VMEM((BLK,), jnp.int32),
            pltpu.VMEM((BLK + 128,), jnp.int32),
            pltpu.VMEM((1, 64), jnp.int32),
            pltpu.VMEM((1, 64), jnp.int32),
            pltpu.VMEM((2, 64), jnp.int32),
            pltpu.VMEM((64, 64), jnp.float32),
            pltpu.VMEM((64, 16), jnp.float32),
            pltpu.VMEM((64, 16), jnp.float32),
            pltpu.VMEM((2, 64, 144), jnp.float32),
            pltpu.VMEM_SHARED((CH + 64, 144), jnp.float32),
            pltpu.SemaphoreType.DMA,
            pltpu.SemaphoreType.DMA,
        ],
    )
    return fn(x, as16, ad16, src_pad, dst_pad)


def _final_body(acc_ref, w_ref, b_ref, out_ref):
    a = acc_ref[...]
    w = w_ref[...]
    outs = []
    for h in range(H):
        d = a[:, 128 + h:129 + h] + 1e-16
        m = a[:, h * C:(h + 1) * C] / d
        y = lax.dot_general(m, w[h * C:(h + 1) * C, :],
                            (((1,), (1,)), ((), ())),
                            preferred_element_type=jnp.float32)
        outs.append(y)
    out_ref[...] = jnp.concatenate(outs, axis=1) + b_ref[...][None, :]


def _final(acc, W, bias):
    grid = (N_PAD // 512,)
    return pl.pallas_call(
        _final_body,
        grid=grid,
        in_specs=[
            pl.BlockSpec((512, 144), lambda i: (i, 0)),
            pl.BlockSpec((128, 64), lambda i: (0, 0)),
            pl.BlockSpec((128,), lambda i: (0,)),
        ],
        out_specs=pl.BlockSpec((512, 128), lambda i: (i, 0)),
        out_shape=jax.ShapeDtypeStruct((N_NODES, H * C), jnp.float32),
    )(acc, W, bias)


def kernel(x, W, att_src, att_dst, bias, edge_index):
    n = x.shape[0]
    loop = jnp.arange(n, dtype=edge_index.dtype)
    src = jnp.concatenate([edge_index[0], loop])
    dst = jnp.concatenate([edge_index[1], loop])
    npad = EPAD - src.shape[0]
    src_pad = jnp.concatenate([src, jnp.zeros((npad,), jnp.int32)])
    dst_pad = jnp.concatenate([dst, jnp.full((npad,), PAD_DST, jnp.int32)])
    as16, ad16 = _prep(x, W, att_src, att_dst)
    acc = _edge_phase(x, as16, ad16, src_pad, dst_pad)
    return _final(acc, W, bias)


# speedup vs baseline: 49.6705x; 1.0004x over previous
import jax
import jax.numpy as jnp
from jax import lax
from jax.experimental import pallas as pl
from jax.experimental.pallas import tpu as pltpu
from jax.experimental.pallas import tpu_sc as plsc

N_NODES = 66125
N_PAD = 66560
H = 2
C = 64

NC, NS = 2, 16
BLK = 5888
NBATCH = BLK // 128
NBLK = 12
EB_SCAN = NBLK * BLK
EPAD = NS * EB_SCAN
CH = 9216
NCHUNK = 8
NACC = CH * NCHUNK
BT = 64
ROWS_T = CH // NS
PAD_DST = 1 << 30


def _prep_body(x_ref, w_ref, asrc_ref, adst_ref, as_ref, ad_ref):
    xb = x_ref[...]
    w = w_ref[...]
    xw = lax.dot_general(xb, w, (((1,), (1,)), ((), ())),
                         preferred_element_type=jnp.float32)
    a_s = asrc_ref[...].reshape(H, C)
    a_d = adst_ref[...].reshape(H, C)
    cols = []
    for tab in (a_s, a_d):
        for h in range(H):
            v = jnp.sum(xw[:, h * C:(h + 1) * C] * tab[h][None, :], axis=1)
            cols.append(v.reshape(-1, 1))
    as_ref[...] = jnp.concatenate([cols[0], cols[1]] * 8, axis=1)
    ad_ref[...] = jnp.concatenate([cols[2], cols[3]] * 8, axis=1)


def _prep(x, W, att_src, att_dst):
    grid = (N_PAD // 512,)
    return pl.pallas_call(
        _prep_body,
        grid=grid,
        in_specs=[
            pl.BlockSpec((512, 64), lambda i: (i, 0)),
            pl.BlockSpec((128, 64), lambda i: (0, 0)),
            pl.BlockSpec((1, H, C), lambda i: (0, 0, 0)),
            pl.BlockSpec((1, H, C), lambda i: (0, 0, 0)),
        ],
        out_specs=[
            pl.BlockSpec((512, 16), lambda i: (i, 0)),
            pl.BlockSpec((512, 16), lambda i: (i, 0)),
        ],
        out_shape=[
            jax.ShapeDtypeStruct((N_PAD, 16), jnp.float32),
            jax.ShapeDtypeStruct((N_PAD, 16), jnp.float32),
        ],
    )(x, W, att_src, att_dst)


def _edge_body(x_hbm, as_hbm, ad_hbm, src_hbm, dst_hbm, acc_hbm,
               src_v, dst_v, cpack, sidx, adidx, sdidx,
               xbuf, asbuf, adbuf, srows, acc_sp, gsem, ssem):
    cid = lax.axis_index("c")
    tid = lax.axis_index("s")
    iota16 = lax.iota(jnp.int32, 16)
    zero16 = jnp.zeros((16,), jnp.float32)
    zero16i = jnp.zeros((16,), jnp.int32)
    MAXI = jnp.int32(0x7FFFFFFF)

    def chunk_body(j, _):
        lo = (cid * 4 + j) * CH

        def zrow(r, _):
            for c in range(9):
                srows[0, r, pl.ds(c * 16, 16)] = zero16
            return 0
        lax.fori_loop(0, 64, zrow, 0)

        def zacc(t, _):
            pltpu.sync_copy(srows.at[0],
                            acc_sp.at[pl.ds(tid * ROWS_T + t * 64, 64), :])
            return 0
        lax.fori_loop(0, ROWS_T // 64, zacc, 0)
        plsc.subcore_barrier()

        def block_body(blk, _):
            base = tid * EB_SCAN + blk * BLK
            pltpu.sync_copy(src_hbm.at[pl.ds(base, BLK)], src_v)
            pltpu.sync_copy(dst_hbm.at[pl.ds(base, BLK)], dst_v)

            def filt(g, k):
                s = src_v[pl.ds(g * 16, 16)]
                d = dst_v[pl.ds(g * 16, 16)]
                valid = (d >= lo) & (d < lo + CH)
                packed = jnp.where(
                    valid, lax.shift_left(s, 14) | (d - lo), MAXI)
                ks, _unused = plsc.sort_key_val(packed, packed)
                cpack[pl.ds(k, 16)] = ks
                return k + plsc.all_reduce_population_count(valid)[0]
            k = lax.fori_loop(0, BLK // 16, filt, jnp.int32(0))
            kp = ((k + 63) // 64) * 64
            nb = kp // 64

            def padv(v, _):
                lane = v * 16 + iota16
                pv = cpack[pl.ds(v * 16, 16)]
                cpack[pl.ds(v * 16, 16)] = jnp.where(lane < k, pv, zero16i)
                return 0
            lax.fori_loop(k // 16, kp // 16, padv, 0)

            def batch_body(b, _):
                par = b % 2

                @pl.when(b >= 2)
                def _():
                    pltpu.make_async_copy(
                        srows.at[0],
                        acc_sp.at[pl.ds(0, 64), :], ssem).wait()

                def cpy(q, _):
                    v = cpack[pl.ds(b * 64 + q * 16, 16)]
                    s = lax.shift_right_logical(v, 14)
                    drel = v & 16383
                    lane = b * 64 + q * 16 + iota16
                    sidx[0, pl.ds(q * 16, 16)] = s
                    adidx[0, pl.ds(q * 16, 16)] = drel + lo
                    sdidx[par, pl.ds(q * 16, 16)] = jnp.where(
                        lane < k, drel, CH)
                    return 0
                lax.fori_loop(0, 4, cpy, 0)

                cg1 = pltpu.async_copy(
                    x_hbm.at[plsc.Indices(sidx.at[0])], xbuf, gsem)
                cg2 = pltpu.async_copy(
                    as_hbm.at[plsc.Indices(sidx.at[0])], asbuf, gsem)
                cg3 = pltpu.async_copy(
                    ad_hbm.at[plsc.Indices(adidx.at[0])], adbuf, gsem)
                cg1.wait()
                cg2.wait()
                cg3.wait()

                def sgrp(q, _):
                    for l in range(16):
                        e = q * 16 + l
                        av = asbuf[e, :] + adbuf[e, :]
                        av = jnp.where(av >= 0.0, av, 0.2 * av)
                        wv = jnp.exp(av)
                        w0 = wv[0]
                        w1 = wv[1]
                        for r in range(4):
                            xv = xbuf[e, pl.ds(r * 16, 16)]
                            srows[par, e, pl.ds(r * 16, 16)] = xv * w0
                            srows[par, e, pl.ds(64 + r * 16, 16)] = xv * w1
                        srows[par, e, pl.ds(128, 16)] = jnp.where(
                            iota16 < 2, wv, 0.0)
                    return 0
                lax.fori_loop(0, 4, sgrp, 0)

                pltpu.async_copy(
                    srows.at[par],
                    acc_sp.at[plsc.Indices(sdidx.at[par])], ssem,
                    add=True)
                return 0
            lax.fori_loop(0, nb, batch_body, 0)

            def drain(i, _):
                pltpu.make_async_copy(
                    srows.at[0], acc_sp.at[pl.ds(0, 64), :], ssem).wait()
                return 0
            lax.fori_loop(0, jnp.minimum(nb, 2), drain, 0)
            return 0
        lax.fori_loop(0, NBLK, block_body, 0)
        plsc.subcore_barrier()

        def dump(t, _):
            r = tid * ROWS_T + t * 64
            pltpu.sync_copy(acc_sp.at[pl.ds(r, 64), :],
                            acc_hbm.at[pl.ds(lo + r, 64), :])
            return 0
        lax.fori_loop(0, ROWS_T // 64, dump, 0)
        return 0
    lax.fori_loop(0, 4, chunk_body, 0)


def _edge_phase(x, as16, ad16, src_pad, dst_pad):
    mesh = plsc.VectorSubcoreMesh(core_axis_name="c", subcore_axis_name="s",
                                  num_cores=NC, num_subcores=NS)
    fn = pl.kernel(
        _edge_body,
        out_type=jax.ShapeDtypeStruct((NACC, 144), jnp.float32),
        mesh=mesh,
        compiler_params=pltpu.CompilerParams(use_tc_tiling_on_sc=False, needs_layout_passes=False),
        scratch_types=[
            pltpu.VMEM((BLK,), jnp.int32),
            pltpu.VMEM((BLK,), jnp.int32),
            pltpu.VMEM((BLK + 128,), jnp.int32),
            pltpu.VMEM((1, 64), jnp.int32),
            pltpu.VMEM((1, 64), jnp.int32),
            pltpu.VMEM((2, 64), jnp.int32),
            pltpu.VMEM((64, 64), jnp.float32),
            pltpu.VMEM((64, 16), jnp.float32),
            pltpu.VMEM((64, 16), jnp.float32),
            pltpu.VMEM((2, 64, 144), jnp.float32),
            pltpu.VMEM_SHARED((CH + 64, 144), jnp.float32),
            pltpu.SemaphoreType.DMA,
            pltpu.SemaphoreType.DMA,
        ],
    )
    return fn(x, as16, ad16, src_pad, dst_pad)


def _final_body(acc_ref, w_ref, b_ref, out_ref):
    a = acc_ref[...]
    w = w_ref[...]
    outs = []
    for h in range(H):
        d = a[:, 128 + h:129 + h] + 1e-16
        m = a[:, h * C:(h + 1) * C] / d
        y = lax.dot_general(m, w[h * C:(h + 1) * C, :],
                            (((1,), (1,)), ((), ())),
                            preferred_element_type=jnp.float32)
        outs.append(y)
    out_ref[...] = jnp.concatenate(outs, axis=1) + b_ref[...][None, :]


def _final(acc, W, bias):
    grid = (N_PAD // 512,)
    return pl.pallas_call(
        _final_body,
        grid=grid,
        in_specs=[
            pl.BlockSpec((512, 144), lambda i: (i, 0)),
            pl.BlockSpec((128, 64), lambda i: (0, 0)),
            pl.BlockSpec((128,), lambda i: (0,)),
        ],
        out_specs=pl.BlockSpec((512, 128), lambda i: (i, 0)),
        out_shape=jax.ShapeDtypeStruct((N_NODES, H * C), jnp.float32),
    )(acc, W, bias)


def kernel(x, W, att_src, att_dst, bias, edge_index):
    n = x.shape[0]
    loop = jnp.arange(n, dtype=edge_index.dtype)
    src = jnp.concatenate([edge_index[0], loop])
    dst = jnp.concatenate([edge_index[1], loop])
    npad = EPAD - src.shape[0]
    src_pad = jnp.concatenate([src, jnp.zeros((npad,), jnp.int32)])
    dst_pad = jnp.concatenate([dst, jnp.full((npad,), PAD_DST, jnp.int32)])
    as16, ad16 = _prep(x, W, att_src, att_dst)
    acc = _edge_phase(x, as16, ad16, src_pad, dst_pad)
    return _final(acc, W, bias)
